# baseline (speedup 1.0000x reference)
import jax
import jax.numpy as jnp
from jax.experimental import pallas as pl
from jax.experimental.pallas import tpu as pltpu

N_TOKENS = 16384
D_MODEL = 4096
N_EXPERTS = 64
TOP_K = 8
BLK = 1024


def _router_body(x_ref, wt_ref, w_out, idx_out, cnt_ref):
    xb = x_ref[...]
    wt = wt_ref[...]
    logits = jax.lax.dot_general(
        xb, wt, (((1,), (0,)), ((), ())),
        preferred_element_type=jnp.float32,
    )

    m = jnp.max(logits, axis=1, keepdims=True)
    e = jnp.exp(logits - m)
    p = e * (1.0 / jnp.sum(e, axis=1, keepdims=True))

    cols = jax.lax.broadcasted_iota(jnp.int32, (BLK, N_EXPERTS), 1)
    bits = jax.lax.bitcast_convert_type(p, jnp.int32)
    packed = (bits & ~0x3F) | (N_EXPERTS - 1 - cols)

    sentinel = jnp.int32(-0x80000000)
    tops = []
    work = packed
    for _ in range(TOP_K):
        mx = jnp.max(work, axis=1, keepdims=True)
        tops.append(mx)
        work = jnp.where(work == mx, sentinel, work)
    top = jnp.concatenate(tops, axis=1)
    idx_out[...] = (N_EXPERTS - 1) - (top & 0x3F)
    w_out[...] = jax.lax.bitcast_convert_type(top & ~0x3F, jnp.float32)

    contrib = jnp.sum((work < 0).astype(jnp.int32), axis=0,
                      keepdims=True)

    @pl.when(pl.program_id(0) == 0)
    def _():
        cnt_ref[...] = jnp.zeros_like(cnt_ref)

    cnt_ref[...] += contrib


def kernel(x, W):
    n, d = x.shape
    e = W.shape[0]
    wt = W.T
    grid = n // BLK
    weights, indices, counts = pl.pallas_call(
        _router_body,
        grid=(grid,),
        in_specs=[
            pl.BlockSpec((BLK, d), lambda i: (i, 0)),
            pl.BlockSpec((d, e), lambda i: (0, 0)),
        ],
        out_specs=[
            pl.BlockSpec((BLK, TOP_K), lambda i: (i, 0)),
            pl.BlockSpec((BLK, TOP_K), lambda i: (i, 0)),
            pl.BlockSpec((1, e), lambda i: (0, 0)),
        ],
        out_shape=[
            jax.ShapeDtypeStruct((n, TOP_K), jnp.float32),
            jax.ShapeDtypeStruct((n, TOP_K), jnp.int32),
            jax.ShapeDtypeStruct((1, e), jnp.int32),
        ],
    )(x, wt)
    return (weights.astype(x.dtype), indices.astype(jnp.int64),
            counts.reshape(e))
